# Initial kernel scaffold; baseline (speedup 1.0000x reference)
import jax
import jax.numpy as jnp
from jax.experimental import pallas as pl


def kernel(x, params, edge_index, batch):
    raise NotImplementedError("write your pallas kernel here")


# speedup vs baseline: 1.0758x; 1.0758x over previous
import functools
import jax
import jax.numpy as jnp
from jax.experimental import pallas as pl

EPS = 1e-5
NEG = -1e30
EBLK = 4000
NBLK = 2000


def _edge1_kernel(hi_ref, hj_ref, wa_ref, wb_ref, y1_ref, s_ref, q_ref):
    i = pl.program_id(0)
    hi = hi_ref[...]
    d = hj_ref[...] - hi
    y1 = (jnp.dot(hi, wa_ref[...], preferred_element_type=jnp.float32)
          + jnp.dot(d, wb_ref[...], preferred_element_type=jnp.float32))
    y1_ref[...] = y1

    @pl.when(i == 0)
    def _():
        s_ref[...] = jnp.zeros_like(s_ref)
        q_ref[...] = jnp.zeros_like(q_ref)

    s_ref[...] += jnp.sum(y1, axis=0, keepdims=True)
    q_ref[...] += jnp.sum(y1 * y1, axis=0, keepdims=True)


def _edge2_kernel(y1_ref, s1_ref, q1_ref, w2_ref, y2_ref, s_ref, q_ref,
                  *, ne):
    i = pl.program_id(0)
    m1 = s1_ref[...] / ne
    v1 = q1_ref[...] / ne - m1 * m1
    sd1 = jnp.sqrt(v1 + EPS)
    a1 = jnp.maximum((y1_ref[...] - m1) / sd1, 0.0)
    y2 = jnp.dot(a1, w2_ref[...], preferred_element_type=jnp.float32)
    y2_ref[...] = y2

    @pl.when(i == 0)
    def _():
        s_ref[...] = jnp.zeros_like(s_ref)
        q_ref[...] = jnp.zeros_like(q_ref)

    s_ref[...] += jnp.sum(y2, axis=0, keepdims=True)
    q_ref[...] += jnp.sum(y2 * y2, axis=0, keepdims=True)


def _node_kernel(mx_ref, s_ref, q_ref, h_ref, *, ne):
    m = s_ref[...] / ne
    v = q_ref[...] / ne - m * m
    sd = jnp.sqrt(v + EPS)
    mx = mx_ref[...]
    h_ref[...] = jnp.where(mx > 0.5 * NEG,
                           jnp.maximum((mx - m) / sd, 0.0), 0.0)


def _head_kernel(gsum_ref, gmax_ref, cnt_ref, fw1_ref, fw2_ref, fw3_ref,
                 out_ref):
    gmean = gsum_ref[...] / jnp.maximum(cnt_ref[...], 1.0)
    feat = jnp.concatenate([gmean, gmax_ref[...]], axis=1)
    h = jnp.dot(feat, fw1_ref[...], preferred_element_type=jnp.float32)
    m = jnp.mean(h, axis=0, keepdims=True)
    c = h - m
    var = jnp.mean(c * c, axis=0, keepdims=True)
    h = jnp.maximum(c / jnp.sqrt(var + EPS), 0.0)
    h = jnp.maximum(
        jnp.dot(h, fw2_ref[...], preferred_element_type=jnp.float32), 0.0)
    logits = jnp.dot(h, fw3_ref[...], preferred_element_type=jnp.float32)
    mx = jnp.max(logits, axis=1, keepdims=True)
    s = logits - mx
    lse = jnp.log(jnp.sum(jnp.exp(s), axis=1, keepdims=True))
    out_ref[...] = s - lse


def _const(shape):
    return pl.BlockSpec(shape, lambda i: (0, 0))


def _edge1_call(hi, hj, wa, wb):
    e, fin = hi.shape
    fout = wa.shape[1]
    espec = lambda w: pl.BlockSpec((EBLK, w), lambda i: (i, 0))
    return pl.pallas_call(
        _edge1_kernel,
        grid=(e // EBLK,),
        in_specs=[espec(fin), espec(fin), _const(wa.shape), _const(wb.shape)],
        out_specs=[espec(fout), _const((1, fout)), _const((1, fout))],
        out_shape=[
            jax.ShapeDtypeStruct((e, fout), jnp.float32),
            jax.ShapeDtypeStruct((1, fout), jnp.float32),
            jax.ShapeDtypeStruct((1, fout), jnp.float32),
        ],
    )(hi, hj, wa, wb)


def _edge2_call(y1, s1, q1, w2, ne):
    e, fin = y1.shape
    fout = w2.shape[1]
    espec = lambda w: pl.BlockSpec((EBLK, w), lambda i: (i, 0))
    return pl.pallas_call(
        functools.partial(_edge2_kernel, ne=ne),
        grid=(e // EBLK,),
        in_specs=[espec(fin), _const((1, fin)), _const((1, fin)),
                  _const(w2.shape)],
        out_specs=[espec(fout), _const((1, fout)), _const((1, fout))],
        out_shape=[
            jax.ShapeDtypeStruct((e, fout), jnp.float32),
            jax.ShapeDtypeStruct((1, fout), jnp.float32),
            jax.ShapeDtypeStruct((1, fout), jnp.float32),
        ],
    )(y1, s1, q1, w2)


def _node_call(mx, s, q, ne):
    n, f = mx.shape
    nspec = pl.BlockSpec((NBLK, f), lambda i: (i, 0))
    return pl.pallas_call(
        functools.partial(_node_kernel, ne=ne),
        grid=(n // NBLK,),
        in_specs=[nspec, _const((1, f)), _const((1, f))],
        out_specs=nspec,
        out_shape=jax.ShapeDtypeStruct((n, f), jnp.float32),
    )(mx, s, q)


def _head_call(gsum, gmax, cnt, fw1, fw2, fw3):
    b = gsum.shape[0]
    return pl.pallas_call(
        _head_kernel,
        grid=(1,),
        in_specs=[_const(gsum.shape), _const(gmax.shape), _const(cnt.shape),
                  _const(fw1.shape), _const(fw2.shape), _const(fw3.shape)],
        out_specs=_const((b, fw3.shape[1])),
        out_shape=jax.ShapeDtypeStruct((b, fw3.shape[1]), jnp.float32),
    )(gsum, gmax, cnt, fw1, fw2, fw3)


def kernel(x, params, edge_index, batch):
    p = params
    n = x.shape[0]
    e = edge_index.shape[1]
    ne = float(e)
    src = edge_index[0]
    dst = edge_index[1]

    def econv2l(h, w1, w2, fin):
        wa, wb = w1[:fin], w1[fin:]
        hi = h[dst]
        hj = h[src]
        y1, s1, q1 = _edge1_call(hi, hj, wa, wb)
        y2, s2, q2 = _edge2_call(y1, s1, q1, w2, ne)
        mx = jnp.full((n, y2.shape[1]), NEG, jnp.float32).at[dst].max(y2)
        return _node_call(mx, s2, q2, ne)

    def econv1l(h, w1, fin):
        wa, wb = w1[:fin], w1[fin:]
        hi = h[dst]
        hj = h[src]
        y1, s1, q1 = _edge1_call(hi, hj, wa, wb)
        mx = jnp.full((n, y1.shape[1]), NEG, jnp.float32).at[dst].max(y1)
        return _node_call(mx, s1, q1, ne)

    h1 = econv2l(x, p["c1w1"], p["c1w2"], 5)
    h2 = econv2l(h1, p["c2w1"], p["c2w2"], 64)
    h3 = econv1l(h2, p["c3w1"], 128)

    b = 64
    counts = jnp.zeros((b,), jnp.float32).at[batch].add(1.0)
    gsum = jnp.zeros((b, 256), jnp.float32).at[batch].add(h3)
    gmax = jnp.zeros((b, 256), jnp.float32).at[batch].max(h3)
    return _head_call(gsum, gmax, counts[:, None], p["fw1"], p["fw2"],
                      p["fw3"])
